# baseline (speedup 1.0000x reference)
import functools

import jax
import jax.numpy as jnp
from jax.experimental import pallas as pl
from jax.experimental.pallas import tpu as pltpu

_BF = jnp.float32


def _conv3x3(pad_ref, a, H, W, w_all, bias, prelu=None, residual=None):
    cin = pad_ref.shape[-1]
    if a is not None:
        pad_ref[1:H + 1, 1:W + 1, :] = a.reshape(H, W, cin).astype(_BF)
    acc = bias.astype(jnp.float32)
    for t in range(9):
        kh, kw = t // 3, t % 3
        patch = pad_ref[kh:kh + H, kw:kw + W, :].reshape(H * W, cin)
        acc = acc + jnp.dot(patch, w_all[t * cin:(t + 1) * cin, :],
                            preferred_element_type=jnp.float32)
    if prelu is not None:
        acc = jnp.where(acc >= 0.0, acc, prelu * acc)
    if residual is not None:
        acc = acc + residual
    return acc


def _net_kernel(x_ref, wsm_ref, bsm_ref, whead_ref, bhead_ref,
                w3_ref, b3_ref, w1_ref, b1_ref, wb_ref, bb_ref,
                wbody_ref, bbody_ref, wup_ref, bup_ref, prelu_ref,
                o_ref, pad_ref, pad3_ref, feat_ref, cur_ref, acc_ref,
                *, H, W, nf, nb):
    blk = pl.program_id(1)
    pad_ref[...] = jnp.zeros_like(pad_ref)

    @pl.when(blk == 0)
    def _():
        x = x_ref[0].reshape(H * W, 3)
        sm = jnp.dot(x, wsm_ref[...], preferred_element_type=jnp.float32)
        sm = sm + bsm_ref[...]
        pad3_ref[...] = jnp.zeros_like(pad3_ref)
        pad3_ref[1:H + 1, 1:W + 1, :] = sm.reshape(H, W, 3).astype(_BF)
        h = bhead_ref[...].astype(jnp.float32)
        for t in range(9):
            kh, kw = t // 3, t % 3
            patch = pad3_ref[kh:kh + H, kw:kw + W, :].reshape(H * W, 3)
            h = h + jnp.dot(patch, whead_ref[t],
                            preferred_element_type=jnp.float32)
        feat_ref[...] = h
        cur_ref[...] = h
        acc_ref[...] = jnp.zeros_like(acc_ref)

    prelu = prelu_ref[...]
    cur = cur_ref[...]
    c3 = [0]
    c1 = [0]

    def conv3(a, act=False, residual=None):
        i = c3[0]
        c3[0] += 1
        return _conv3x3(pad_ref, a, H, W, w3_ref[0, i], b3_ref[0, i],
                        prelu=prelu if act else None, residual=residual)

    def conv1(chunks):
        i = c1[0]
        c1[0] += 1
        out = b1_ref[0, i].astype(jnp.float32)
        for k, ck in enumerate(chunks):
            out = out + jnp.dot(ck.astype(_BF), w1_ref[0, i, k * nf:(k + 1) * nf, :],
                                preferred_element_type=jnp.float32)
        return out

    def rcb(a):
        return conv3(conv3(a, act=True), residual=a)

    def fract2(a):
        return [rcb(rcb(a)), rcb(a)]

    def fract4(a):
        ch = fract2(a)
        ch2 = fract2(conv1(ch))
        return ch2 + [rcb(a)]

    def fract8(a):
        ch = fract4(a)
        ch2 = fract4(conv1(ch))
        return ch2 + [rcb(a)]

    res = conv3(conv1(fract8(cur)), residual=cur)
    cur_ref[...] = res
    acc_ref[...] = acc_ref[...] + jnp.dot(res.astype(_BF), wb_ref[0],
                                          preferred_element_type=jnp.float32)

    @pl.when(blk == nb - 1)
    def _():
        bottle = acc_ref[...] + bb_ref[...]
        body = _conv3x3(pad_ref, bottle, H, W, wbody_ref[...], bbody_ref[...],
                        residual=feat_ref[...])
        up = _conv3x3(pad_ref, body, H, W, wup_ref[...], bup_ref[...])
        o_ref[0] = up.reshape(H, W, 4 * nf).astype(o_ref.dtype)


def _build_net(N, H, W, nf, nb):
    body = functools.partial(_net_kernel, H=H, W=W, nf=nf, nb=nb)
    return pl.pallas_call(
        body,
        out_shape=jax.ShapeDtypeStruct((N, H, W, 4 * nf), jnp.float32),
        grid=(N, nb),
        in_specs=[
            pl.BlockSpec((1, H, W, 3), lambda n, i: (n, 0, 0, 0)),
            pl.BlockSpec((3, 3), lambda n, i: (0, 0)),
            pl.BlockSpec((1, 3), lambda n, i: (0, 0)),
            pl.BlockSpec((9, 3, nf), lambda n, i: (0, 0, 0)),
            pl.BlockSpec((1, nf), lambda n, i: (0, 0)),
            pl.BlockSpec((1, 31, 9 * nf, nf), lambda n, i: (i, 0, 0, 0)),
            pl.BlockSpec((1, 31, 1, nf), lambda n, i: (i, 0, 0, 0)),
            pl.BlockSpec((1, 4, 4 * nf, nf), lambda n, i: (i, 0, 0, 0)),
            pl.BlockSpec((1, 4, 1, nf), lambda n, i: (i, 0, 0, 0)),
            pl.BlockSpec((1, nf, nf), lambda n, i: (i, 0, 0)),
            pl.BlockSpec((1, nf), lambda n, i: (0, 0)),
            pl.BlockSpec((9 * nf, nf), lambda n, i: (0, 0)),
            pl.BlockSpec((1, nf), lambda n, i: (0, 0)),
            pl.BlockSpec((9 * nf, 4 * nf), lambda n, i: (0, 0)),
            pl.BlockSpec((1, 4 * nf), lambda n, i: (0, 0)),
            pl.BlockSpec((1, 1), lambda n, i: (0, 0)),
        ],
        out_specs=pl.BlockSpec((1, H, W, 4 * nf), lambda n, i: (n, 0, 0, 0)),
        scratch_shapes=[
            pltpu.VMEM((H + 2, W + 2, nf), _BF),
            pltpu.VMEM((H + 2, W + 2, 3), _BF),
            pltpu.VMEM((H * W, nf), jnp.float32),
            pltpu.VMEM((H * W, nf), jnp.float32),
            pltpu.VMEM((H * W, nf), jnp.float32),
        ],
        compiler_params=pltpu.CompilerParams(
            dimension_semantics=("parallel", "arbitrary")),
    )


def _tail_kernel(t_ref, wt_ref, bt_ref, wam_ref, bam_ref, o_ref, pad_ref, *, H, W, nf):
    pad_ref[...] = jnp.zeros_like(pad_ref)
    y = _conv3x3(pad_ref, t_ref[0].reshape(H * W, nf), H, W,
                 wt_ref[...], bt_ref[...])
    y = jnp.dot(y, wam_ref[...], preferred_element_type=jnp.float32) + bam_ref[...]
    o_ref[0] = y.reshape(H, W, 3).astype(o_ref.dtype)


def _build_tail(N, H, W, nf):
    body = functools.partial(_tail_kernel, H=H, W=W, nf=nf)
    return pl.pallas_call(
        body,
        out_shape=jax.ShapeDtypeStruct((N, H, W, 3), jnp.float32),
        grid=(N,),
        in_specs=[
            pl.BlockSpec((1, H, W, nf), lambda n: (n, 0, 0, 0)),
            pl.BlockSpec((9 * nf, 3), lambda n: (0, 0)),
            pl.BlockSpec((1, 3), lambda n: (0, 0)),
            pl.BlockSpec((3, 3), lambda n: (0, 0)),
            pl.BlockSpec((1, 3), lambda n: (0, 0)),
        ],
        out_specs=pl.BlockSpec((1, H, W, 3), lambda n: (n, 0, 0, 0)),
        scratch_shapes=[pltpu.VMEM((H + 2, W + 2, nf), _BF)],
        compiler_params=pltpu.CompilerParams(dimension_semantics=("parallel",)),
    )


def _pixel_shuffle(x, r):
    N, H, W, C = x.shape
    c = C // (r * r)
    x = x.reshape(N, H, W, c, r, r)
    x = jnp.transpose(x, (0, 1, 4, 2, 5, 3))
    return x.reshape(N, H * r, W * r, c)


def kernel(x, w_sm, b_sm, w_am, b_am, w_head, b_head, w3, b3, w1, b1, wb, bb,
           w_body, b_body, w_up, b_up, w_tail, b_tail, prelu):
    N, H, W, _ = x.shape
    nf = b_head.shape[1]
    nb = w3.shape[0]
    scale = 2

    whead_p = w_head.astype(_BF)
    w3_p = w3.reshape(nb, 31, 9 * nf, nf).astype(_BF)
    w1_p = w1.astype(_BF)
    wb_p = wb.astype(_BF)
    wbody_p = w_body.reshape(9 * nf, nf).astype(_BF)
    wup_p = w_up.reshape(9 * nf, 4 * nf).astype(_BF)
    wt_p = w_tail.reshape(9 * nf, 3).astype(_BF)

    up = _build_net(N, H, W, nf, nb)(
        x, w_sm, b_sm, whead_p, b_head, w3_p, b3, w1_p, b1, wb_p, bb,
        wbody_p, b_body, wup_p, b_up, prelu)

    t = _pixel_shuffle(up, scale)
    return _build_tail(N, H * scale, W * scale, nf)(t, wt_p, b_tail, w_am, b_am)
